# baseline (speedup 1.0000x reference)
import functools

import jax
import jax.numpy as jnp
from jax import lax
from jax.experimental import pallas as pl
from jax.experimental.pallas import tpu as pltpu
from jax.experimental.pallas import tpu_sc as plsc

DIM = 128
LANES = 16
ND = DIM // LANES
NC = 2
NS = 16
NW = NC * NS
BATCH = 16384
B_PER_W = BATCH // NW
CHUNK = 64
N_CHUNKS = B_PER_W // CHUNK


def _make_sc_kernel():
    mesh = plsc.VectorSubcoreMesh(core_axis_name="c", subcore_axis_name="s")

    row_t = pltpu.VMEM((CHUNK, DIM), jnp.float32)

    @functools.partial(
        pl.kernel,
        mesh=mesh,
        out_type=jax.ShapeDtypeStruct((NW, LANES), jnp.float32),
        scratch_types=(
            [pltpu.VMEM((6 * B_PER_W,), jnp.int32)]
            + [pltpu.VMEM((6, B_PER_W), jnp.int32)]
            + [row_t] * 6
            + [row_t] * 6
            + [pltpu.VMEM((LANES,), jnp.float32),
               pltpu.SemaphoreType.DMA,
               pltpu.SemaphoreType.DMA,
               pltpu.VMEM_SHARED((1000, DIM), jnp.float32),
               pltpu.VMEM_SHARED((1000, DIM), jnp.float32)]
        ),
    )
    def dist_mult(pt_hbm, nt_hbm, ent_hbm, rel_hbm, out_hbm, *scratch):
        trip_v = scratch[0]
        idx_v = scratch[1]
        row_a = scratch[2:8]
        row_b = scratch[8:14]
        out_v, sem_a, sem_b = scratch[14], scratch[15], scratch[16]
        ent_s, rel_s = scratch[17], scratch[18]

        tables = (ent_s, rel_s, ent_s, ent_s, rel_s, ent_s)

        cid = lax.axis_index("c")
        sid = lax.axis_index("s")
        wid = sid * NC + cid
        base = wid * B_PER_W

        iota = jnp.arange(LANES, dtype=jnp.int32)
        rots = [((iota + k) & (LANES - 1))[:, None] for k in (8, 4, 2, 1)]
        dnums = lax.GatherDimensionNumbers(
            offset_dims=(), collapsed_slice_dims=(0,), start_index_map=(0,))

        def hsum(v):
            for r in rots:
                v = v + lax.gather(
                    v, r, dnums, slice_sizes=(1,),
                    mode=lax.GatherScatterMode.PROMISE_IN_BOUNDS)
            return v

        @pl.when(sid == 0)
        def _():
            pltpu.sync_copy(ent_hbm.at[pl.ds(0, 1000)], ent_s)
            pltpu.sync_copy(rel_hbm, rel_s)

        pltpu.sync_copy(pt_hbm.at[wid], trip_v.at[pl.ds(0, 3 * B_PER_W)])
        pltpu.sync_copy(nt_hbm.at[wid],
                        trip_v.at[pl.ds(3 * B_PER_W, 3 * B_PER_W)])

        def dg(v, idx):
            return lax.gather(v, idx[:, None], dnums, slice_sizes=(1,),
                              mode=lax.GatherScatterMode.PROMISE_IN_BOUNDS)

        shuf = []
        for col in range(3):
            i0 = (3 * iota + col) & 15
            i1 = (3 * iota + col - 16) & 15
            i2 = (3 * iota + col - 32) & 15
            m0 = (3 * iota + col) < 16
            m1 = (3 * iota + col) < 32
            shuf.append((i0, i1, i2, m0, m1))

        for blk in range(2):
            for g in range(B_PER_W // LANES):
                fb = blk * 3 * B_PER_W + g * 3 * LANES
                v0 = trip_v[pl.ds(fb, LANES)]
                v1 = trip_v[pl.ds(fb + LANES, LANES)]
                v2 = trip_v[pl.ds(fb + 2 * LANES, LANES)]
                for col in range(3):
                    i0, i1, i2, m0, m1 = shuf[col]
                    ids = jnp.where(
                        m0, dg(v0, i0), jnp.where(m1, dg(v1, i1), dg(v2, i2)))
                    idx_v[3 * blk + col, pl.ds(g * LANES, LANES)] = ids

        plsc.subcore_barrier()

        def issue(ci, rows, sem):
            for j, (tab, r) in enumerate(zip(tables, rows)):
                ib = idx_v.at[j, pl.ds(ci * CHUNK, CHUNK)]
                pltpu.async_copy(tab.at[ib], r, sem)

        def drain(ci, rows, sem):
            for j, (tab, r) in enumerate(zip(tables, rows)):
                ib = idx_v.at[j, pl.ds(ci * CHUNK, CHUNK)]
                pltpu.make_async_copy(tab.at[ib], r, sem).wait()

        def compute(rows, tot):
            hp_v, rp_v, tp_v, hn_v, rn_v, tn_v = rows

            def pair_body(i, t):
                s0 = pl.ds(0, LANES)
                accp = hp_v[i, s0] * rp_v[i, s0] * tp_v[i, s0]
                accn = hn_v[i, s0] * rn_v[i, s0] * tn_v[i, s0]
                for d in range(1, ND):
                    s = pl.ds(d * LANES, LANES)
                    accp = accp + hp_v[i, s] * rp_v[i, s] * tp_v[i, s]
                    accn = accn + hn_v[i, s] * rn_v[i, s] * tn_v[i, s]
                diff = hsum(accp - accn)
                return t + jnp.maximum(diff + 1.0, 0.0)

            return lax.fori_loop(0, CHUNK, pair_body, tot)

        issue(0, row_a, sem_a)

        def body(k, tot):
            issue(2 * k + 1, row_b, sem_b)
            drain(2 * k, row_a, sem_a)
            tot = compute(row_a, tot)

            nxt = 2 * k + 2

            @pl.when(nxt < N_CHUNKS)
            def _():
                issue(nxt, row_a, sem_a)

            drain(2 * k + 1, row_b, sem_b)
            return compute(row_b, tot)

        total = lax.fori_loop(0, N_CHUNKS // 2, body,
                              jnp.zeros((LANES,), jnp.float32))
        out_v[...] = total
        pltpu.sync_copy(out_v, out_hbm.at[wid])

    return dist_mult


_dist_mult = _make_sc_kernel()


@jax.jit
def kernel(positive_triples, negative_triples, entities, relations):
    pt = positive_triples.astype(jnp.int32).reshape(NW, 3 * B_PER_W)
    nt = negative_triples.astype(jnp.int32).reshape(NW, 3 * B_PER_W)
    partials = _dist_mult(pt, nt, entities, relations)
    return jnp.sum(partials[:, 0]) / jnp.float32(BATCH)
